# baseline (speedup 1.0000x reference)
import functools

import jax
import jax.numpy as jnp
from jax import lax
from jax.experimental import pallas as pl
from jax.experimental.pallas import tpu as pltpu
from jax.experimental.pallas import tpu_sc as plsc

C = 64
L = 16
NB = 4
IR = 8


def _make_sc_conv(N, D, E_pad):
    info = plsc.get_sparse_core_info()
    NC, NS = info.num_cores, info.num_subcores
    NW = NC * NS
    n_chunks = E_pad // C
    cpt = n_chunks // NW
    assert n_chunks % NW == 0 and cpt % IR == 0 and N % L == 0

    RC = C
    n_full = N // RC
    tail = N - n_full * RC
    n_row_chunks = n_full + (1 if tail else 0)
    jj_iters = -(-n_row_chunks // NS)

    mesh = plsc.VectorSubcoreMesh(core_axis_name="c", subcore_axis_name="s")

    @functools.partial(
        pl.kernel,
        out_type=(
            jax.ShapeDtypeStruct((NC, N, D), jnp.float32),
            jax.ShapeDtypeStruct((NW * N,), jnp.float32),
        ),
        mesh=mesh,
        compiler_params=pltpu.CompilerParams(needs_layout_passes=False),
        scratch_types=[
            [pltpu.VMEM((C,), jnp.int32)] * IR,
            [pltpu.VMEM((C,), jnp.int32)] * IR,
            [pltpu.VMEM((C,), jnp.float32)] * IR,
            [pltpu.VMEM((C, D), jnp.float32)] * NB,
            pltpu.VMEM((N,), jnp.float32),
            pltpu.VMEM_SHARED((N, D), jnp.float32),
            [pltpu.SemaphoreType.DMA] * IR,
            [pltpu.SemaphoreType.DMA] * NB,
            [pltpu.SemaphoreType.DMA] * NB,
        ],
    )
    def conv(x_hbm, src_hbm, dst_hbm, ea_hbm, y_hbm, s_hbm,
             src_b, dst_b, ea_b, rows, s_acc, y_sh, isem, gsem, ssem):
        bounce = rows[0]
        cid = lax.axis_index("c")
        sid = lax.axis_index("s")
        wid = sid * NC + cid
        my_base = wid * cpt

        zero16 = jnp.zeros((L,), jnp.float32)

        def zrow(r, _):
            for k in range(D // L):
                bounce[r, pl.ds(k * L, L)] = zero16
            return 0
        lax.fori_loop(0, RC, zrow, 0)

        for jj in range(jj_iters):
            ch = sid + jj * NS
            rr = pl.multiple_of(ch * RC, RC)

            @pl.when(ch < n_full)
            def _():
                pltpu.sync_copy(bounce, y_sh.at[pl.ds(rr, RC)])
            if tail:
                @pl.when(ch == n_full)
                def _():
                    pltpu.sync_copy(bounce.at[pl.ds(0, tail)],
                                    y_sh.at[pl.ds(n_full * RC, tail)])

        def zs(i, _):
            s_acc[pl.ds(i * L, L)] = zero16
            return 0
        lax.fori_loop(0, N // L, zs, 0)

        plsc.subcore_barrier()

        def i_start(c, r):
            eb = pl.multiple_of((my_base + c) * C, 8)
            pltpu.async_copy(src_hbm.at[pl.ds(eb, C)], src_b[r], isem[r])
            pltpu.async_copy(dst_hbm.at[pl.ds(eb, C)], dst_b[r], isem[r])
            pltpu.async_copy(ea_hbm.at[pl.ds(eb, C)], ea_b[r], isem[r])

        def i_wait(r):
            pltpu.make_async_copy(src_hbm.at[pl.ds(0, C)], src_b[r], isem[r]).wait()
            pltpu.make_async_copy(dst_hbm.at[pl.ds(0, C)], dst_b[r], isem[r]).wait()
            pltpu.make_async_copy(ea_hbm.at[pl.ds(0, C)], ea_b[r], isem[r]).wait()

        def g_start(r, b):
            pltpu.async_copy(x_hbm.at[src_b[r]], rows[b], gsem[b])

        def g_wait(b):
            pltpu.make_async_copy(x_hbm.at[pl.ds(0, C)], rows[b], gsem[b]).wait()

        def s_start(r, b):
            pltpu.async_copy(rows[b], y_sh.at[dst_b[r]], ssem[b], add=True)

        def s_wait(b):
            pltpu.make_async_copy(rows[b], y_sh.at[pl.ds(0, C)], ssem[b]).wait()

        for c in range(NB):
            i_start(c, c)
        i_wait(0)
        g_start(0, 0)

        n_outer = cpt // IR

        def outer(o, _):
            for b8 in range(IR):
                b = b8 % NB
                i = o * IR + b8

                g_wait(b)

                if b8 >= 3:
                    s_wait((b + 1) % NB)
                else:
                    @pl.when(o >= 1)
                    def _():
                        s_wait((b + 1) % NB)

                if b8 < NB:
                    i_start(i + NB, (b8 + NB) % IR)
                else:
                    @pl.when(o < n_outer - 1)
                    def _():
                        i_start(i + NB, (b8 + NB) % IR)

                def _g():
                    i_wait((b8 + 1) % IR)
                    g_start((b8 + 1) % IR, (b + 1) % NB)
                if b8 == IR - 1:
                    @pl.when(o < n_outer - 1)
                    def _():
                        _g()
                else:
                    _g()

                sb, eb_, rb = src_b[b8], ea_b[b8], rows[b]

                def scale16(j, _):
                    src16 = sb[pl.ds(j * L, L)]
                    ea16 = eb_[pl.ds(j * L, L)]
                    plsc.addupdate_scatter(s_acc, [src16], ea16)
                    for t in range(L):
                        e = j * L + t
                        bc = plsc.load_gather(eb_, [jnp.full((L,), e, jnp.int32)])
                        for k in range(D // L):
                            rb[e, pl.ds(k * L, L)] = rb[e, pl.ds(k * L, L)] * bc
                    return 0
                lax.fori_loop(0, C // L, scale16, 0)

                s_start(b8, b)
            return 0
        lax.fori_loop(0, n_outer, outer, 0)

        for k in range(1, NB):
            s_wait((cpt - NB + k) % NB)

        plsc.subcore_barrier()

        for jj in range(jj_iters):
            ch = sid + jj * NS
            rr = pl.multiple_of(ch * RC, RC)

            @pl.when(ch < n_full)
            def _():
                pltpu.sync_copy(y_sh.at[pl.ds(rr, RC)], bounce)
                pltpu.sync_copy(bounce, y_hbm.at[cid, pl.ds(rr, RC)])
            if tail:
                @pl.when(ch == n_full)
                def _():
                    pltpu.sync_copy(y_sh.at[pl.ds(n_full * RC, tail)],
                                    bounce.at[pl.ds(0, tail)])
                    pltpu.sync_copy(bounce.at[pl.ds(0, tail)],
                                    y_hbm.at[cid, pl.ds(n_full * RC, tail)])
        pltpu.sync_copy(s_acc, s_hbm.at[pl.ds(pl.multiple_of(wid * N, 8), N)])

    return conv


def _tc_dense(y_part, s_part, x, ea_self, W1, b1, W2, b2):
    N, D = x.shape
    H = W1.shape[0]

    def body(yp, sp, xb, eas, W1r, b1r, W2r, b2r, out):
        y = yp[0] + yp[1] + eas[...] * xb[...]
        h1 = lax.dot_general(y, W1r[...], (((1,), (1,)), ((), ())),
                             preferred_element_type=jnp.float32)
        h1 = jnp.maximum(h1 + b1r[...], 0.0)
        stot = jnp.sum(sp[...], axis=0)[:, None] + eas[...]
        v = jnp.sum(stot * h1, axis=0, keepdims=True) * (1.0 / N)
        out[...] = lax.dot_general(v, W2r[...], (((1,), (1,)), ((), ())),
                                   preferred_element_type=jnp.float32) + b2r[...]

    return pl.pallas_call(
        body,
        out_shape=jax.ShapeDtypeStruct((1, H), jnp.float32),
    )(y_part, s_part, x, ea_self, W1, b1.reshape(1, H), W2, b2.reshape(1, H))


def kernel(x, edge_index, edge_attr, W1, b1, W2, b2):
    N, D = x.shape
    E = edge_index.shape[1]

    info = plsc.get_sparse_core_info()
    NW = info.num_cores * info.num_subcores

    unit = C * NW * IR
    E_pad = -(-E // unit) * unit
    pad = E_pad - E
    src = jnp.concatenate([edge_index[0], jnp.zeros((pad,), jnp.int32)])
    dst = jnp.concatenate([edge_index[1], jnp.zeros((pad,), jnp.int32)])
    ea_e = jnp.concatenate([edge_attr[:E], jnp.zeros((pad,), jnp.float32)])
    ea_self = edge_attr[E:].reshape(N, 1)

    conv = _make_sc_conv(N, D, E_pad)
    y_part, s_flat = conv(x, src, dst, ea_e)
    s_part = s_flat.reshape(NW, N)
    out = _tc_dense(y_part, s_part, x, ea_self, W1, b1, W2, b2)
    return out.reshape(D)
